# Initial kernel scaffold; baseline (speedup 1.0000x reference)
import jax
import jax.numpy as jnp
from jax.experimental import pallas as pl


def kernel(x, edge_index, W1, b1, W2, b2, W3, b3):
    raise NotImplementedError("write your pallas kernel here")


# speedup vs baseline: 36.9376x; 36.9376x over previous
import functools

import jax
import jax.numpy as jnp
from jax import lax
from jax.experimental import pallas as pl
from jax.experimental.pallas import tpu as pltpu
from jax.experimental.pallas import tpu_sc as plsc

N = 10000
E = 320000
D_IN = 128
H1 = 256
H2 = 32
C = 7
CPAD = 16

NSC = 2
NTL = 16
NW = NSC * NTL
EPT = E // NW
CH = 100
NCH = EPT // CH
NP = 10240
RPT = NP // NTL
ZR = 128

_MESH = dict(core_axis_name="c", subcore_axis_name="s",
             num_cores=NSC, num_subcores=NTL)


def _fill(ref, rows, width, value):
    vals = jnp.full((16,), value, jnp.float32)

    def body(i, _):
        for j in range(width // 16):
            ref[i, pl.ds(j * 16, 16)] = vals
        return 0

    lax.fori_loop(0, rows, body, 0)


def _deg_body(dst_hbm, out_hbm, acc, idx_d, ones):
    cid = lax.axis_index("c")
    sid = lax.axis_index("s")
    tid = cid * NTL + sid
    _fill(ones, CH, 16, 0.0)
    for i in range(RPT // CH):
        pltpu.sync_copy(ones, acc.at[pl.ds(sid * RPT + i * CH, CH)])
    rem = RPT - (RPT // CH) * CH
    pltpu.sync_copy(ones.at[pl.ds(0, rem)],
                    acc.at[pl.ds(sid * RPT + RPT - rem, rem)])
    _fill(ones, CH, 16, 1.0)
    pltpu.sync_copy(dst_hbm.at[tid], idx_d)
    plsc.subcore_barrier()

    def chunk(c, _):
        pltpu.sync_copy(ones, acc.at[idx_d.at[c]], add=True)
        return 0

    lax.fori_loop(0, NCH, chunk, 0)
    plsc.subcore_barrier()
    pltpu.sync_copy(acc.at[pl.ds(sid * RPT, RPT)],
                    out_hbm.at[cid, pl.ds(sid * RPT, RPT)])


_deg_kernel = functools.partial(
    pl.kernel,
    out_type=jax.ShapeDtypeStruct((NSC, NP, 16), jnp.float32),
    mesh=plsc.VectorSubcoreMesh(**_MESH),
    compiler_params=pltpu.CompilerParams(use_tc_tiling_on_sc=False),
    scratch_types=[
        pltpu.VMEM_SHARED((NP, 16), jnp.float32),
        pltpu.VMEM((NCH, CH), jnp.int32),
        pltpu.VMEM((CH, 16), jnp.float32),
    ],
)(_deg_body)


def _make_agg(F):
    def body(src_hbm, dst_hbm, xs_hbm, out_hbm,
             acc, idx_s, idx_d, rows_a, rows_b, sem_a, sem_b):
        cid = lax.axis_index("c")
        sid = lax.axis_index("s")
        tid = cid * NTL + sid
        _fill(rows_a, CH, F, 0.0)
        for i in range(RPT // CH):
            pltpu.sync_copy(rows_a, acc.at[pl.ds(sid * RPT + i * CH, CH)])
        rem = RPT - (RPT // CH) * CH
        pltpu.sync_copy(rows_a.at[pl.ds(0, rem)],
                        acc.at[pl.ds(sid * RPT + RPT - rem, rem)])
        pltpu.sync_copy(src_hbm.at[tid], idx_s)
        pltpu.sync_copy(dst_hbm.at[tid], idx_d)
        plsc.subcore_barrier()

        pltpu.async_copy(xs_hbm.at[idx_s.at[0]], rows_a, sem_a)
        pltpu.async_copy(xs_hbm.at[idx_s.at[1]], rows_b, sem_b)

        def pair(i, _):
            c = 2 * i
            pltpu.make_async_copy(xs_hbm.at[idx_s.at[c]], rows_a, sem_a).wait()
            pltpu.sync_copy(rows_a, acc.at[idx_d.at[c]], add=True)
            na = jnp.minimum(c + 2, NCH - 1)
            pltpu.async_copy(xs_hbm.at[idx_s.at[na]], rows_a, sem_a)
            pltpu.make_async_copy(xs_hbm.at[idx_s.at[c + 1]], rows_b, sem_b).wait()
            pltpu.sync_copy(rows_b, acc.at[idx_d.at[c + 1]], add=True)
            nb = jnp.minimum(c + 3, NCH - 1)
            pltpu.async_copy(xs_hbm.at[idx_s.at[nb]], rows_b, sem_b)
            return 0

        lax.fori_loop(0, NCH // 2, pair, 0)
        pltpu.make_async_copy(xs_hbm.at[idx_s.at[0]], rows_a, sem_a).wait()
        pltpu.make_async_copy(xs_hbm.at[idx_s.at[0]], rows_b, sem_b).wait()
        plsc.subcore_barrier()
        pltpu.sync_copy(acc.at[pl.ds(sid * RPT, RPT)],
                        out_hbm.at[cid, pl.ds(sid * RPT, RPT)])

    return functools.partial(
        pl.kernel,
        out_type=jax.ShapeDtypeStruct((NSC, NP, F), jnp.float32),
        mesh=plsc.VectorSubcoreMesh(**_MESH),
        compiler_params=pltpu.CompilerParams(use_tc_tiling_on_sc=False),
        scratch_types=[
            pltpu.VMEM_SHARED((NP, F), jnp.float32),
            pltpu.VMEM((NCH, CH), jnp.int32),
            pltpu.VMEM((NCH, CH), jnp.int32),
            pltpu.VMEM((CH, F), jnp.float32),
            pltpu.VMEM((CH, F), jnp.float32),
            pltpu.SemaphoreType.DMA,
            pltpu.SemaphoreType.DMA,
        ],
    )(body)


_agg128 = _make_agg(D_IN)
_agg32 = _make_agg(H2)
_agg16 = _make_agg(CPAD)


def _tc1_body(degp_ref, x_ref, dis_ref, xs_ref):
    degp = degp_ref[...]
    deg = degp[0, :N, 0:1] + degp[1, :N, 0:1] + 1.0
    dis = lax.rsqrt(deg)
    dis_ref[...] = dis
    xs_ref[...] = x_ref[...] * dis


_tc1 = pl.pallas_call(
    _tc1_body,
    out_shape=[
        jax.ShapeDtypeStruct((N, 1), jnp.float32),
        jax.ShapeDtypeStruct((N, D_IN), jnp.float32),
    ],
)


def _tc2_body(p_ref, xs_ref, dis_ref, W1_ref, b1_ref, W2_ref, t2s_ref):
    p = p_ref[...]
    dis = dis_ref[...]
    agg1 = (p[0, :N] + p[1, :N] + xs_ref[...]) * dis
    h1 = jnp.dot(agg1, W1_ref[...], preferred_element_type=jnp.float32)
    h1 = jnp.maximum(h1 + b1_ref[...][None, :], 0.0)
    t2 = jnp.dot(h1, W2_ref[...], preferred_element_type=jnp.float32)
    t2s_ref[...] = t2 * dis


_tc2 = pl.pallas_call(
    _tc2_body,
    out_shape=jax.ShapeDtypeStruct((N, H2), jnp.float32),
)


def _tc3_body(q_ref, t2s_ref, dis_ref, b2_ref, W3p_ref, t3s_ref):
    q = q_ref[...]
    dis = dis_ref[...]
    agg2 = (q[0, :N] + q[1, :N] + t2s_ref[...]) * dis
    h2 = jnp.maximum(agg2 + b2_ref[...][None, :], 0.0)
    t3 = jnp.dot(h2, W3p_ref[...], preferred_element_type=jnp.float32)
    t3s_ref[...] = t3 * dis


_tc3 = pl.pallas_call(
    _tc3_body,
    out_shape=jax.ShapeDtypeStruct((N, CPAD), jnp.float32),
)


def _tc4_body(r_ref, t3s_ref, dis_ref, b3_ref, out_ref):
    r = r_ref[...]
    h3p = (r[0, :N] + r[1, :N] + t3s_ref[...]) * dis_ref[...]
    h3 = h3p[:, :C] + b3_ref[...][None, :]
    m = jnp.max(h3, axis=1, keepdims=True)
    e = jnp.exp(h3 - m)
    lse = jnp.log(jnp.sum(e, axis=1, keepdims=True)) + m
    out_ref[...] = h3 - lse


_tc4 = pl.pallas_call(
    _tc4_body,
    out_shape=jax.ShapeDtypeStruct((N, C), jnp.float32),
)


def kernel(x, edge_index, W1, b1, W2, b2, W3, b3):
    src = edge_index[0].reshape(NW, NCH, CH)
    dst = edge_index[1].reshape(NW, NCH, CH)
    W3p = jnp.pad(W3, ((0, 0), (0, CPAD - C)))

    degp = _deg_kernel(dst)
    dis, xs = _tc1(degp, x)
    p1 = _agg128(src, dst, xs)
    t2s = _tc2(p1, xs, dis, W1, b1, W2)
    p2 = _agg32(src, dst, t2s)
    t3s = _tc3(p2, t2s, dis, b2, W3p)
    p3 = _agg16(src, dst, t3s)
    return _tc4(p3, t3s, dis, b3)
